# baseline (speedup 1.0000x reference)
import functools

import jax
import jax.numpy as jnp
from jax import lax
from jax.experimental import pallas as pl
from jax.experimental.pallas import tpu as pltpu
from jax.experimental.pallas import tpu_sc as plsc

B = 16384
V_DIM = 64
A_DIM = 64
HIDDEN = 128
PAIR_DIM = 2 * V_DIM
NUM_ROWS = 100000

_info = plsc.get_sparse_core_info()
NC = _info.num_cores
NS = _info.num_subcores
NW = NC * NS
B_PER_W = B // NW
IDX_CHUNK = 128
CHUNKS = B_PER_W // IDX_CHUNK

HALF_ROWS = NUM_ROWS // 2
PACK_BLK = 1000
NBLK_HALF = HALF_ROWS // PACK_BLK


def _pack_body(lo_ref, hi_ref, out_ref):
    out_ref[...] = jnp.concatenate([lo_ref[...], hi_ref[...]], axis=1)


def _pack_table(table):
    return pl.pallas_call(
        _pack_body,
        grid=(NBLK_HALF,),
        in_specs=[
            pl.BlockSpec((PACK_BLK, V_DIM), lambda i: (i, 0)),
            pl.BlockSpec((PACK_BLK, V_DIM), lambda i: (i + NBLK_HALF, 0)),
        ],
        out_specs=pl.BlockSpec((PACK_BLK, PAIR_DIM), lambda i: (i, 0)),
        out_shape=jax.ShapeDtypeStruct((HALF_ROWS, PAIR_DIM), jnp.float32),
        compiler_params=pltpu.CompilerParams(
            dimension_semantics=("arbitrary",),
        ),
    )(table, table)


_sc_mesh = plsc.VectorSubcoreMesh(core_axis_name="c", subcore_axis_name="s")


@functools.partial(
    pl.kernel,
    mesh=_sc_mesh,
    out_type=jax.ShapeDtypeStruct((B, PAIR_DIM), jnp.float32),
    scratch_types=[
        pltpu.VMEM((CHUNKS, IDX_CHUNK), jnp.int32),
        pltpu.VMEM((B_PER_W, PAIR_DIM), jnp.float32),
        pltpu.SemaphoreType.DMA,
    ],
)
def _sc_gather(idx_hbm, table_hbm, out_hbm, idx_v, rows_v, sem):
    wid = lax.axis_index("s") * NC + lax.axis_index("c")
    pltpu.sync_copy(idx_hbm.at[pl.ds(wid * CHUNKS, CHUNKS)], idx_v)
    for j in range(CHUNKS):
        pltpu.make_async_copy(
            table_hbm.at[idx_v.at[j]],
            rows_v.at[pl.ds(j * IDX_CHUNK, IDX_CHUNK)],
            sem,
        ).start()
    for j in range(CHUNKS):
        pltpu.make_async_copy(
            table_hbm.at[idx_v.at[j]],
            rows_v.at[pl.ds(j * IDX_CHUNK, IDX_CHUNK)],
            sem,
        ).wait()
    pltpu.sync_copy(rows_v, out_hbm.at[pl.ds(wid * B_PER_W, B_PER_W)])


ROWS_BLK = 1024


def _tc_body(raw_ref, idx_ref, age_ref, g_ref, bt_ref, w1_ref, b1_ref,
             w2_ref, b2_ref, out_ref):
    raw = raw_ref[...]
    sel = idx_ref[...].reshape(ROWS_BLK, 1) >= HALF_ROWS
    v = jnp.where(sel, raw[:, V_DIM:], raw[:, :V_DIM])
    mu = jnp.mean(v, axis=-1, keepdims=True)
    c = v - mu
    var = jnp.mean(c * c, axis=-1, keepdims=True)
    v_feat = c * lax.rsqrt(var + 1e-5) * g_ref[...] + bt_ref[...]
    a = age_ref[...].reshape(ROWS_BLK, 1)
    h = jnp.maximum(a * w1_ref[...] + b1_ref[...], 0.0)
    af = jnp.dot(h, w2_ref[...], preferred_element_type=jnp.float32)
    a_feat = jnp.maximum(af + b2_ref[...], 0.0)
    out_ref[...] = jnp.concatenate([v_feat, a_feat], axis=1)


def _tc_fuse(raw, idx, age, g2, bt2, W1, b1_2, W2, b2_2):
    grid = (B // ROWS_BLK,)
    return pl.pallas_call(
        _tc_body,
        grid=grid,
        in_specs=[
            pl.BlockSpec((ROWS_BLK, PAIR_DIM), lambda i: (i, 0)),
            pl.BlockSpec((ROWS_BLK,), lambda i: (i,)),
            pl.BlockSpec((ROWS_BLK,), lambda i: (i,)),
            pl.BlockSpec((1, V_DIM), lambda i: (0, 0)),
            pl.BlockSpec((1, V_DIM), lambda i: (0, 0)),
            pl.BlockSpec((1, HIDDEN), lambda i: (0, 0)),
            pl.BlockSpec((1, HIDDEN), lambda i: (0, 0)),
            pl.BlockSpec((HIDDEN, A_DIM), lambda i: (0, 0)),
            pl.BlockSpec((1, A_DIM), lambda i: (0, 0)),
        ],
        out_specs=pl.BlockSpec((ROWS_BLK, V_DIM + A_DIM), lambda i: (i, 0)),
        out_shape=jax.ShapeDtypeStruct((B, V_DIM + A_DIM), jnp.float32),
        compiler_params=pltpu.CompilerParams(
            dimension_semantics=("arbitrary",),
        ),
    )(raw, idx, age, g2, bt2, W1, b1_2, W2, b2_2)


def kernel(variety_idx, age, table, ln_gamma, ln_beta, W1, b1, W2, b2):
    idx = variety_idx.astype(jnp.int32)
    idx_pair = jnp.where(idx < HALF_ROWS, idx, idx - HALF_ROWS)
    idx_pair = idx_pair.reshape(NW * CHUNKS, IDX_CHUNK)
    table128 = _pack_table(table)
    raw = _sc_gather(idx_pair, table128)
    g2 = ln_gamma.reshape(1, V_DIM)
    bt2 = ln_beta.reshape(1, V_DIM)
    b1_2 = b1.reshape(1, HIDDEN)
    b2_2 = b2.reshape(1, A_DIM)
    return _tc_fuse(raw, idx, age, g2, bt2, W1, b1_2, W2, b2_2)


# speedup vs baseline: 1.1760x; 1.1760x over previous
import functools

import jax
import jax.numpy as jnp
from jax import lax
from jax.experimental import pallas as pl
from jax.experimental.pallas import tpu as pltpu
from jax.experimental.pallas import tpu_sc as plsc

B = 16384
V_DIM = 64
A_DIM = 64
HIDDEN = 128

_info = plsc.get_sparse_core_info()
NC = _info.num_cores
NS = _info.num_subcores
NW = NC * NS
B_PER_W = B // NW
IDX_CHUNK = 128
CHUNKS = B_PER_W // IDX_CHUNK

_sc_mesh = plsc.VectorSubcoreMesh(core_axis_name="c", subcore_axis_name="s")


@functools.partial(
    pl.kernel,
    mesh=_sc_mesh,
    out_type=jax.ShapeDtypeStruct((B, V_DIM), jnp.float32),
    scratch_types=[
        pltpu.VMEM((CHUNKS, IDX_CHUNK), jnp.int32),
        pltpu.VMEM((B_PER_W, V_DIM), jnp.float32),
        pltpu.SemaphoreType.DMA,
    ],
    compiler_params=pltpu.CompilerParams(use_tc_tiling_on_sc=False),
)
def _sc_gather(idx_hbm, table_hbm, out_hbm, idx_v, rows_v, sem):
    wid = lax.axis_index("s") * NC + lax.axis_index("c")
    pltpu.sync_copy(idx_hbm.at[pl.ds(wid * CHUNKS, CHUNKS)], idx_v)
    for j in range(CHUNKS):
        pltpu.make_async_copy(
            table_hbm.at[idx_v.at[j]],
            rows_v.at[pl.ds(j * IDX_CHUNK, IDX_CHUNK)],
            sem,
        ).start()
    for j in range(CHUNKS):
        pltpu.make_async_copy(
            table_hbm.at[idx_v.at[j]],
            rows_v.at[pl.ds(j * IDX_CHUNK, IDX_CHUNK)],
            sem,
        ).wait()
    pltpu.sync_copy(rows_v, out_hbm.at[pl.ds(wid * B_PER_W, B_PER_W)])


ROWS_BLK = 1024


def _tc_body(raw_ref, age_ref, g_ref, bt_ref, w1_ref, b1_ref, w2_ref, b2_ref,
             out_ref):
    v = raw_ref[...]
    mu = jnp.mean(v, axis=-1, keepdims=True)
    c = v - mu
    var = jnp.mean(c * c, axis=-1, keepdims=True)
    v_feat = c * lax.rsqrt(var + 1e-5) * g_ref[...] + bt_ref[...]
    a = age_ref[...].reshape(ROWS_BLK, 1)
    h = jnp.maximum(a * w1_ref[...] + b1_ref[...], 0.0)
    af = jnp.dot(h, w2_ref[...], preferred_element_type=jnp.float32)
    a_feat = jnp.maximum(af + b2_ref[...], 0.0)
    out_ref[...] = jnp.concatenate([v_feat, a_feat], axis=1)


def _tc_fuse(raw, age, g2, bt2, W1, b1_2, W2, b2_2):
    grid = (B // ROWS_BLK,)
    return pl.pallas_call(
        _tc_body,
        grid=grid,
        in_specs=[
            pl.BlockSpec((ROWS_BLK, V_DIM), lambda i: (i, 0)),
            pl.BlockSpec((ROWS_BLK,), lambda i: (i,)),
            pl.BlockSpec((1, V_DIM), lambda i: (0, 0)),
            pl.BlockSpec((1, V_DIM), lambda i: (0, 0)),
            pl.BlockSpec((1, HIDDEN), lambda i: (0, 0)),
            pl.BlockSpec((1, HIDDEN), lambda i: (0, 0)),
            pl.BlockSpec((HIDDEN, A_DIM), lambda i: (0, 0)),
            pl.BlockSpec((1, A_DIM), lambda i: (0, 0)),
        ],
        out_specs=pl.BlockSpec((ROWS_BLK, V_DIM + A_DIM), lambda i: (i, 0)),
        out_shape=jax.ShapeDtypeStruct((B, V_DIM + A_DIM), jnp.float32),
        compiler_params=pltpu.CompilerParams(
            dimension_semantics=("arbitrary",),
        ),
    )(raw, age, g2, bt2, W1, b1_2, W2, b2_2)


def kernel(variety_idx, age, table, ln_gamma, ln_beta, W1, b1, W2, b2):
    idx = variety_idx.astype(jnp.int32).reshape(NW * CHUNKS, IDX_CHUNK)
    raw = _sc_gather(idx, table)
    g2 = ln_gamma.reshape(1, V_DIM)
    bt2 = ln_beta.reshape(1, V_DIM)
    b1_2 = b1.reshape(1, HIDDEN)
    b2_2 = b2.reshape(1, A_DIM)
    return _tc_fuse(raw, age, g2, bt2, W1, b1_2, W2, b2_2)


# speedup vs baseline: 1.2286x; 1.0448x over previous
import functools

import jax
import jax.numpy as jnp
from jax import lax
from jax.experimental import pallas as pl
from jax.experimental.pallas import tpu as pltpu
from jax.experimental.pallas import tpu_sc as plsc

B = 16384
V_DIM = 64
A_DIM = 64
HIDDEN = 128

_info = plsc.get_sparse_core_info()
NC = _info.num_cores
NS = _info.num_subcores
NW = NC * NS
B_PER_W = B // NW
IDX_CHUNK = 128
CHUNKS = B_PER_W // IDX_CHUNK

_sc_mesh = plsc.VectorSubcoreMesh(core_axis_name="c", subcore_axis_name="s")


@functools.partial(
    pl.kernel,
    mesh=_sc_mesh,
    out_type=jax.ShapeDtypeStruct((B, V_DIM), jnp.float32),
    scratch_types=[
        pltpu.VMEM((CHUNKS, IDX_CHUNK), jnp.int32),
        pltpu.VMEM((B_PER_W, V_DIM), jnp.float32),
        pltpu.SemaphoreType.DMA,
    ],
    compiler_params=pltpu.CompilerParams(use_tc_tiling_on_sc=False),
)
def _sc_gather(idx_hbm, table_hbm, out_hbm, idx_v, rows_v, sem):
    wid = lax.axis_index("s") * NC + lax.axis_index("c")
    pltpu.sync_copy(idx_hbm.at[pl.ds(wid * CHUNKS, CHUNKS)], idx_v)
    for j in range(CHUNKS):
        pltpu.make_async_copy(
            table_hbm.at[idx_v.at[j]],
            rows_v.at[pl.ds(j * IDX_CHUNK, IDX_CHUNK)],
            sem,
        ).start()
    for j in range(CHUNKS):
        pltpu.make_async_copy(
            table_hbm.at[idx_v.at[j]],
            rows_v.at[pl.ds(j * IDX_CHUNK, IDX_CHUNK)],
            sem,
        ).wait()
    pltpu.sync_copy(rows_v, out_hbm.at[pl.ds(wid * B_PER_W, B_PER_W)])


ROWS_BLK = 2048


def _tc_body(raw_ref, age_ref, g_ref, bt_ref, w1_ref, b1_ref, w2_ref, b2_ref,
             out_ref):
    v = raw_ref[...]
    mu = jnp.mean(v, axis=-1, keepdims=True)
    c = v - mu
    var = jnp.mean(c * c, axis=-1, keepdims=True)
    v_feat = c * lax.rsqrt(var + 1e-5) * g_ref[...] + bt_ref[...]
    a = age_ref[...].reshape(ROWS_BLK, 1)
    h = jnp.maximum(a * w1_ref[...] + b1_ref[...], 0.0)
    af = jnp.dot(h, w2_ref[...], preferred_element_type=jnp.float32)
    a_feat = jnp.maximum(af + b2_ref[...], 0.0)
    out_ref[...] = jnp.concatenate([v_feat, a_feat], axis=1)


def _tc_fuse(raw, age, g2, bt2, W1, b1_2, W2, b2_2):
    grid = (B // ROWS_BLK,)
    return pl.pallas_call(
        _tc_body,
        grid=grid,
        in_specs=[
            pl.BlockSpec((ROWS_BLK, V_DIM), lambda i: (i, 0)),
            pl.BlockSpec((ROWS_BLK,), lambda i: (i,)),
            pl.BlockSpec((1, V_DIM), lambda i: (0, 0)),
            pl.BlockSpec((1, V_DIM), lambda i: (0, 0)),
            pl.BlockSpec((1, HIDDEN), lambda i: (0, 0)),
            pl.BlockSpec((1, HIDDEN), lambda i: (0, 0)),
            pl.BlockSpec((HIDDEN, A_DIM), lambda i: (0, 0)),
            pl.BlockSpec((1, A_DIM), lambda i: (0, 0)),
        ],
        out_specs=pl.BlockSpec((ROWS_BLK, V_DIM + A_DIM), lambda i: (i, 0)),
        out_shape=jax.ShapeDtypeStruct((B, V_DIM + A_DIM), jnp.float32),
        compiler_params=pltpu.CompilerParams(
            dimension_semantics=("parallel",),
        ),
    )(raw, age, g2, bt2, W1, b1_2, W2, b2_2)


def kernel(variety_idx, age, table, ln_gamma, ln_beta, W1, b1, W2, b2):
    idx = variety_idx.astype(jnp.int32).reshape(NW * CHUNKS, IDX_CHUNK)
    raw = _sc_gather(idx, table)
    g2 = ln_gamma.reshape(1, V_DIM)
    bt2 = ln_beta.reshape(1, V_DIM)
    b1_2 = b1.reshape(1, HIDDEN)
    b2_2 = b2.reshape(1, A_DIM)
    return _tc_fuse(raw, age, g2, bt2, W1, b1_2, W2, b2_2)
